# Initial kernel scaffold; baseline (speedup 1.0000x reference)
import jax
import jax.numpy as jnp
from jax.experimental import pallas as pl


def kernel(hidden_states, router_weight):
    raise NotImplementedError("write your pallas kernel here")


# speedup vs baseline: 1.1331x; 1.1331x over previous
import functools

import jax
import jax.numpy as jnp
import numpy as np
from jax import lax
from jax.experimental import pallas as pl
from jax.experimental.pallas import tpu as pltpu
from jax.experimental.pallas import tpu_sc as plsc

S, B, H, E, TOPK = 8192, 4, 4096, 64, 8
T = S * B
BLK = 1024

NC, NS, L = 2, 16, 16
NW = NC * NS
CHUNK = T // NW
GROUPS = CHUNK // L


def _router_block(x_ref, w_ref, logits_ref, aff_ref, logits_t_ref):
    l = jnp.dot(x_ref[...], w_ref[...], preferred_element_type=jnp.float32)
    logits_ref[...] = l
    m = jnp.max(l, axis=1, keepdims=True)
    e = jnp.exp(l - m)
    aff_ref[...] = e / jnp.sum(e, axis=1, keepdims=True)
    logits_t_ref[...] = l.T


def _dense_stage(x, w):
    z = np.int32(0)
    return pl.pallas_call(
        _router_block,
        grid=(T // BLK,),
        in_specs=[
            pl.BlockSpec((BLK, H), lambda i: (i, z)),
            pl.BlockSpec((H, E), lambda i: (z, z)),
        ],
        out_specs=[
            pl.BlockSpec((BLK, E), lambda i: (i, z)),
            pl.BlockSpec((BLK, E), lambda i: (i, z)),
            pl.BlockSpec((E, BLK), lambda i: (z, i)),
        ],
        out_shape=[
            jax.ShapeDtypeStruct((T, E), jnp.float32),
            jax.ShapeDtypeStruct((T, E), jnp.float32),
            jax.ShapeDtypeStruct((E, T), jnp.float32),
        ],
    )(x, w)


def _topk_body(lt_hbm, out_hbm, lt_v, out_v):
    wid = lax.axis_index("s") * jnp.int32(NC) + lax.axis_index("c")
    base = wid * jnp.int32(CHUNK)
    pltpu.sync_copy(lt_hbm.at[:, pl.ds(base, CHUNK)], lt_v)

    lanes = lax.iota(jnp.int32, L)

    def group(g, carry):
        t0 = g * jnp.int32(L)
        best = [jnp.full((L,), -jnp.inf, jnp.float32) for _ in range(TOPK)]
        bidx = [jnp.zeros((L,), jnp.int32) for _ in range(TOPK)]
        for e in range(E):
            cv = lt_v[e, pl.ds(t0, L)]
            ci = jnp.full((L,), e, jnp.int32)
            for j in range(TOPK):
                m = cv > best[j]
                nb = jnp.where(m, cv, best[j])
                ni = jnp.where(m, ci, bidx[j])
                cv = jnp.where(m, best[j], cv)
                ci = jnp.where(m, bidx[j], ci)
                best[j] = nb
                bidx[j] = ni
        pos0 = (t0 + lanes) * jnp.int32(TOPK)
        for j in range(TOPK):
            plsc.store_scatter(out_v, [pos0 + jnp.int32(j)], bidx[j])
        return carry

    lax.fori_loop(jnp.int32(0), jnp.int32(GROUPS), group, None)
    pltpu.sync_copy(out_v, out_hbm.at[pl.ds(base * jnp.int32(TOPK), CHUNK * TOPK)])


@functools.cache
def _topk_stage():
    return pl.kernel(
        _topk_body,
        mesh=plsc.VectorSubcoreMesh(core_axis_name="c", subcore_axis_name="s"),
        out_type=jax.ShapeDtypeStruct((T * TOPK,), jnp.int32),
        scratch_types=[
            pltpu.VMEM((E, CHUNK), jnp.float32),
            pltpu.VMEM((CHUNK * TOPK,), jnp.int32),
        ],
        compiler_params=pltpu.CompilerParams(needs_layout_passes=False),
    )


def kernel(hidden_states, router_weight):
    x = hidden_states.reshape(T, H)
    logits, aff, logits_t = _dense_stage(x, router_weight)
    expert_index = _topk_stage()(logits_t).reshape(T, TOPK)
    return logits, aff.astype(jnp.float64), expert_index
